# baseline (speedup 1.0000x reference)
import functools
import math

import jax
import jax.numpy as jnp
from jax import lax
from jax.experimental import pallas as pl
from jax.experimental.pallas import tpu as pltpu
from jax.experimental.pallas import tpu_sc as plsc

D_MODEL = 64
SCALE = math.sqrt(D_MODEL)
NC, NS, LANES = 2, 16, 16
NW = NC * NS

W = 128
SUB = 8
NBUF = 4


def _sc_embed(xT, lut_p):
    S, B0 = xT.shape
    n_blk = B0 // W
    n_tasks = S * n_blk
    tpw = n_tasks // NW
    mesh = plsc.VectorSubcoreMesh(core_axis_name="c", subcore_axis_name="s")

    @functools.partial(
        pl.kernel,
        out_type=jax.ShapeDtypeStruct((S, D_MODEL, B0), jnp.float32),
        mesh=mesh,
        compiler_params=pltpu.CompilerParams(needs_layout_passes=False),
        scratch_types=[
            pltpu.VMEM((NBUF, W), jnp.int32),
            pltpu.VMEM((NBUF, W, W), jnp.float32),
            pltpu.VMEM((NBUF, D_MODEL // SUB, SUB, W), jnp.float32),
            [pltpu.SemaphoreType.DMA] * NBUF,
            [pltpu.SemaphoreType.DMA] * NBUF,
        ],
    )
    def k(x_hbm, lut_hbm, out_hbm, idx_v, rows_v, blk_v, gsems, wsems):
        wid = lax.axis_index("s") * NC + lax.axis_index("c")
        base = wid * tpw

        iota = lax.iota(jnp.int32, LANES)
        d_hi = [2 * j + lax.shift_right_logical(iota, 3) for j in range(4)]
        d_lo = lax.bitwise_and(iota, SUB - 1)

        def task_sb(t):
            task = base + t
            return task // n_blk, task % n_blk

        def start_task(t, b):
            s, ib = task_sb(t)
            pltpu.sync_copy(x_hbm.at[s, pl.ds(ib * W, W)], idx_v.at[b])
            pltpu.make_async_copy(
                lut_hbm.at[idx_v.at[b]], rows_v.at[b], gsems[b]
            ).start()

        def wait_gather(t, b):
            s, ib = task_sb(t)
            del s, ib
            pltpu.make_async_copy(
                lut_hbm.at[idx_v.at[b]], rows_v.at[b], gsems[b]
            ).wait()

        def write_descs(t, b):
            s, ib = task_sb(t)
            return [
                pltpu.make_async_copy(
                    blk_v.at[b, dh],
                    out_hbm.at[s, pl.ds(dh * SUB, SUB), pl.ds(ib * W, W)],
                    wsems[b],
                )
                for dh in range(D_MODEL // SUB)
            ]

        for b in range(2):
            start_task(b, b)

        @pl.loop(0, tpw // NBUF)
        def _(tt):
            t0 = tt * NBUF
            for b in range(NBUF):
                t = t0 + b
                pn = (b + 2) % NBUF

                @pl.when(t + 2 < tpw)
                def _():
                    @pl.when(t >= 2)
                    def _():
                        for d in write_descs(t - 2, pn):
                            d.wait()

                    start_task(t + 2, pn)

                wait_gather(t, b)

                @pl.loop(0, W)
                def _(i):
                    il = jnp.full((LANES,), i, jnp.int32)
                    for j in range(4):
                        v = rows_v[b, i, pl.ds(j * LANES, LANES)] * SCALE
                        plsc.store_scatter(
                            blk_v.at[b], [d_hi[j], d_lo, il], v
                        )

                for d in write_descs(t, b):
                    d.start()

        for d in write_descs(tpw - 2, (tpw - 2) % NBUF):
            d.wait()
        for d in write_descs(tpw - 1, (tpw - 1) % NBUF):
            d.wait()

    return k(xT, lut_p)


def kernel(x, lut):
    B0, S = x.shape
    xT = jnp.transpose(x)
    lut_p = jnp.pad(lut, ((0, 0), (0, W - D_MODEL)))
    outT = _sc_embed(xT, lut_p)
    return jnp.transpose(outT, (2, 0, 1))


# speedup vs baseline: 1.3873x; 1.3873x over previous
import functools
import math

import jax
import jax.numpy as jnp
from jax import lax
from jax.experimental import pallas as pl
from jax.experimental.pallas import tpu as pltpu
from jax.experimental.pallas import tpu_sc as plsc

D_MODEL = 64
SCALE = math.sqrt(D_MODEL)
NC, NS, LANES = 2, 16, 16
NW = NC * NS

W = 128
SUB = 8
NBUF = 4


def _sc_embed(xT, lut_p):
    S, B0 = xT.shape
    n_blk = B0 // W
    bpw = n_blk // NW
    tpw = S * bpw
    mesh = plsc.VectorSubcoreMesh(core_axis_name="c", subcore_axis_name="s")

    @functools.partial(
        pl.kernel,
        out_type=jax.ShapeDtypeStruct((S, D_MODEL, B0), jnp.float32),
        mesh=mesh,
        compiler_params=pltpu.CompilerParams(needs_layout_passes=False),
        scratch_types=[
            pltpu.VMEM((S, bpw, W), jnp.int32),
            pltpu.VMEM((NBUF, W, W), jnp.float32),
            pltpu.VMEM((NBUF, D_MODEL // SUB, SUB, W), jnp.float32),
            [pltpu.SemaphoreType.DMA] * NBUF,
            [pltpu.SemaphoreType.DMA] * NBUF,
        ],
    )
    def k(x_hbm, lut_hbm, out_hbm, idx_v, rows_v, blk_v, gsems, wsems):
        wid = lax.axis_index("s") * NC + lax.axis_index("c")

        for ibl in range(bpw):
            pltpu.sync_copy(
                x_hbm.at[:, pl.ds((wid * bpw + ibl) * W, W)],
                idx_v.at[:, ibl],
            )

        iota = lax.iota(jnp.int32, LANES)
        d_hi = [2 * j + lax.shift_right_logical(iota, 3) for j in range(4)]
        d_lo = lax.bitwise_and(iota, SUB - 1)

        def task_sb(t):
            return t // bpw, t % bpw

        def gather_desc(t, b):
            s, ibl = task_sb(t)
            return pltpu.make_async_copy(
                lut_hbm.at[idx_v.at[s, ibl]], rows_v.at[b], gsems[b]
            )

        def write_descs(t, b):
            s, ibl = task_sb(t)
            col = (wid * bpw + ibl) * W
            return [
                pltpu.make_async_copy(
                    blk_v.at[b, dh],
                    out_hbm.at[s, pl.ds(dh * SUB, SUB), pl.ds(col, W)],
                    wsems[b],
                )
                for dh in range(D_MODEL // SUB)
            ]

        for b in range(2):
            gather_desc(b, b).start()

        @pl.loop(0, tpw // NBUF)
        def _(tt):
            t0 = tt * NBUF
            for b in range(NBUF):
                t = t0 + b
                pn = (b + 2) % NBUF

                @pl.when(t + 2 < tpw)
                def _():
                    @pl.when(t >= 2)
                    def _():
                        for d in write_descs(t - 2, pn):
                            d.wait()

                    gather_desc(t + 2, pn).start()

                gather_desc(t, b).wait()

                @plsc.parallel_loop(0, W)
                def _(i):
                    il = jnp.full((LANES,), i, jnp.int32)
                    for j in range(4):
                        v = rows_v[b, i, pl.ds(j * LANES, LANES)] * SCALE
                        plsc.store_scatter(
                            blk_v.at[b], [d_hi[j], d_lo, il], v
                        )

                for d in write_descs(t, b):
                    d.start()

        for d in write_descs(tpw - 2, (tpw - 2) % NBUF):
            d.wait()
        for d in write_descs(tpw - 1, (tpw - 1) % NBUF):
            d.wait()

    return k(xT, lut_p)


def kernel(x, lut):
    B0, S = x.shape
    xT = jnp.transpose(x)
    lut_p = jnp.pad(lut, ((0, 0), (0, W - D_MODEL)))
    outT = _sc_embed(xT, lut_p)
    return jnp.transpose(outT, (2, 0, 1))


# speedup vs baseline: 1.4207x; 1.0241x over previous
import functools
import math

import jax
import jax.numpy as jnp
from jax import lax
from jax.experimental import pallas as pl
from jax.experimental.pallas import tpu as pltpu
from jax.experimental.pallas import tpu_sc as plsc

D_MODEL = 64
SCALE = math.sqrt(D_MODEL)
NC, NS, LANES = 2, 16, 16
NW = NC * NS

W = 128
SUB = 8
NBUF = 4


def _sc_embed(xT, lut_p):
    S, B0 = xT.shape
    n_blk = B0 // W
    bpw = n_blk // NW
    tpw = S * bpw
    mesh = plsc.VectorSubcoreMesh(core_axis_name="c", subcore_axis_name="s")

    @functools.partial(
        pl.kernel,
        out_type=jax.ShapeDtypeStruct((S, D_MODEL, B0), jnp.float32),
        mesh=mesh,
        compiler_params=pltpu.CompilerParams(needs_layout_passes=False),
        scratch_types=[
            pltpu.VMEM((S, bpw, W), jnp.int32),
            pltpu.VMEM((NBUF, W, W), jnp.float32),
            pltpu.VMEM((NBUF, D_MODEL // SUB, SUB, W), jnp.float32),
            [pltpu.SemaphoreType.DMA] * NBUF,
            [pltpu.SemaphoreType.DMA] * NBUF,
        ],
    )
    def k(x_hbm, lut_hbm, out_hbm, idx_v, rows_v, blk_v, gsems, wsems):
        wid = lax.axis_index("s") * NC + lax.axis_index("c")

        for ibl in range(bpw):
            pltpu.sync_copy(
                x_hbm.at[:, pl.ds((wid * bpw + ibl) * W, W)],
                idx_v.at[:, ibl],
            )

        iota = lax.iota(jnp.int32, LANES)
        d_hi = [2 * j + lax.shift_right_logical(iota, 3) for j in range(4)]
        d_lo = lax.bitwise_and(iota, SUB - 1)

        def task_sb(t):
            return t // bpw, t % bpw

        def gather_desc(t, b):
            s, ibl = task_sb(t)
            return pltpu.make_async_copy(
                lut_hbm.at[idx_v.at[s, ibl]], rows_v.at[b], gsems[b]
            )

        def write_descs(t, b):
            s, ibl = task_sb(t)
            col = (wid * bpw + ibl) * W
            return [
                pltpu.make_async_copy(
                    blk_v.at[b, dh],
                    out_hbm.at[s, pl.ds(dh * SUB, SUB), pl.ds(col, W)],
                    wsems[b],
                )
                for dh in range(D_MODEL // SUB)
            ]

        for b in range(2):
            gather_desc(b, b).start()

        @pl.loop(0, tpw // NBUF)
        def _(tt):
            t0 = tt * NBUF
            for b in range(NBUF):
                t = t0 + b
                pn = (b + 2) % NBUF

                @pl.when(t + 2 < tpw)
                def _():
                    @pl.when(t >= 2)
                    def _():
                        for d in write_descs(t - 2, pn):
                            d.wait()

                    gather_desc(t + 2, pn).start()

                gather_desc(t, b).wait()

                @plsc.parallel_loop(0, W, unroll=8)
                def _(i):
                    il = jnp.full((LANES,), i, jnp.int32)
                    for j in range(4):
                        v = rows_v[b, i, pl.ds(j * LANES, LANES)] * SCALE
                        plsc.store_scatter(
                            blk_v.at[b], [d_hi[j], d_lo, il], v
                        )

                for d in write_descs(t, b):
                    d.start()

        for d in write_descs(tpw - 2, (tpw - 2) % NBUF):
            d.wait()
        for d in write_descs(tpw - 1, (tpw - 1) % NBUF):
            d.wait()

    return k(xT, lut_p)


def kernel(x, lut):
    B0, S = x.shape
    xT = jnp.transpose(x)
    lut_p = jnp.pad(lut, ((0, 0), (0, W - D_MODEL)))
    outT = _sc_embed(xT, lut_p)
    return jnp.transpose(outT, (2, 0, 1))
